# baseline (speedup 1.0000x reference)
import functools

import jax
import jax.numpy as jnp
from jax import lax
from jax.experimental import pallas as pl


def _conv_body(x_ref, w_ref, b_ref, o_ref, *, groups, BH, W, Cin, Cout, act,
               transpose_out, pretiled):
    accs = []
    for taps in groups:
        parts = []
        for (t, dy, dx) in taps:
            if pretiled:
                xs = x_ref[0, 0, pl.ds(dy, BH), pl.ds(dx, W), :]
            else:
                r0 = pl.program_id(1) * BH
                xs = x_ref[0, pl.ds(r0 + dy, BH), pl.ds(dx, W), :]
            parts.append(xs.reshape(BH * W, Cin).astype(jnp.bfloat16))
        xcat = parts[0] if len(parts) == 1 else jnp.concatenate(parts, axis=1)
        k0 = taps[0][0] * Cin
        wg = w_ref[pl.ds(k0, len(taps) * Cin), :]
        acc = jnp.dot(xcat, wg, preferred_element_type=jnp.float32)
        acc = acc + b_ref[0]
        if act == "relu":
            acc = jnp.maximum(acc, 0.0)
        elif act == "tanh":
            acc = jnp.tanh(acc)
        accs.append(acc)
    out = accs[0] if len(accs) == 1 else jnp.concatenate(accs, axis=1)
    G = len(groups)
    if transpose_out:
        o_ref[0] = out.T.reshape(G * Cout, BH, W)
    else:
        o_ref[0] = out.reshape(BH, W, G * Cout)


def _conv(xp, w_taps, bias, groups, H, W, act, BH, transpose_out=False,
          halo=0):
    T, Cin, Cout = w_taps.shape
    G = len(groups)
    N = xp.shape[0]
    n_h = H // BH
    body = functools.partial(_conv_body, groups=groups, BH=BH, W=W, Cin=Cin,
                             Cout=Cout, act=act, transpose_out=transpose_out,
                             pretiled=halo > 0)
    if halo == 0:
        _, Hp, Wp, _ = xp.shape
        x_spec = pl.BlockSpec((1, Hp, Wp, Cin), lambda n, h: (n, 0, 0, 0))
    else:
        _, _, Hw, Wp, _ = xp.shape
        x_spec = pl.BlockSpec((1, 1, Hw, Wp, Cin),
                              lambda n, h: (n, h, 0, 0, 0))
    if transpose_out:
        o_spec = pl.BlockSpec((1, G * Cout, BH, W), lambda n, h: (n, 0, h, 0))
        o_shape = jax.ShapeDtypeStruct((N, G * Cout, H, W), jnp.float32)
    else:
        o_spec = pl.BlockSpec((1, BH, W, G * Cout), lambda n, h: (n, h, 0, 0))
        o_shape = jax.ShapeDtypeStruct((N, H, W, G * Cout), jnp.float32)
    return pl.pallas_call(
        body,
        grid=(N, n_h),
        in_specs=[
            x_spec,
            pl.BlockSpec((T * Cin, Cout), lambda n, h: (0, 0)),
            pl.BlockSpec((1, Cout), lambda n, h: (0, 0)),
        ],
        out_specs=o_spec,
        out_shape=o_shape,
    )(xp, w_taps.reshape(T * Cin, Cout).astype(jnp.bfloat16),
      bias.reshape(1, Cout))


def _pad_hw(x, p):
    return jnp.pad(x, ((0, 0), (p, p), (p, p), (0, 0)))


def _space_to_depth2(xp):
    N, H2, W2, C = xp.shape
    U, V = H2 // 2, W2 // 2
    return xp.reshape(N, U, 2, V, 2, C).transpose(0, 1, 3, 2, 4, 5).reshape(
        N, U, V, 4 * C)


def _conv_s1(x_nhwc, w_oikk, bias, act):
    O, C, k, _ = w_oikk.shape
    xp = _pad_hw(x_nhwc, 1)
    H, W = x_nhwc.shape[1], x_nhwc.shape[2]
    w_taps = w_oikk.transpose(2, 3, 1, 0).reshape(k * k, C, O)
    groups = [[(ki * k + kj, ki, kj) for ki in range(k) for kj in range(k)]]
    return _conv(xp, w_taps, bias, groups, H, W, act, BH=min(H, 56))


def _conv_s2(x_nhwc, w_oikk, bias, act):
    O, C, _, _ = w_oikk.shape
    N, H, W, _ = x_nhwc.shape
    Ho, Wo = H // 2, W // 2
    s2d = _space_to_depth2(_pad_hw(x_nhwc, 1))
    wr = w_oikk.reshape(O, C, 2, 2, 2, 2)
    w_taps = wr.transpose(2, 4, 3, 5, 1, 0).reshape(4, 4 * C, O)
    groups = [[(a * 2 + b, a, b) for a in range(2) for b in range(2)]]
    return _conv(s2d, w_taps, bias, groups, Ho, Wo, act, BH=min(Ho, 56))


def _deconv_weights_groups(w_iokk):
    w_list = []
    groups = []
    for p in range(2):
        for q in range(2):
            taps = []
            for a in range(2):
                for b in range(2):
                    t = len(w_list)
                    w_list.append(w_iokk[:, :, 3 - p - 2 * a, 3 - q - 2 * b])
                    taps.append((t, p + a, q + b))
            groups.append(taps)
    return jnp.stack(w_list, axis=0), groups


def _deconv_s2(x_nhwc, w_iokk, bias, act, BH):
    Cin, Cout, _, _ = w_iokk.shape
    N, H, W, _ = x_nhwc.shape
    xp = _pad_hw(x_nhwc, 1)
    w_taps, groups = _deconv_weights_groups(w_iokk)
    out = _conv(xp, w_taps, bias, groups, H, W, act, BH=BH)
    out = out.reshape(N, H, W, 2, 2, Cout).transpose(0, 1, 3, 2, 4, 5)
    return out.reshape(N, 2 * H, 2 * W, Cout)


def _deconv_s2_final(x_nhwc, w_iokk, bias, act, BH):
    Cin, Cout, _, _ = w_iokk.shape
    N, H, W, _ = x_nhwc.shape
    xp = _pad_hw(x_nhwc, 1)
    n_h = H // BH
    xt = jnp.stack([xp[:, i * BH:i * BH + BH + 2] for i in range(n_h)],
                   axis=1)
    w_taps, groups = _deconv_weights_groups(w_iokk)
    out = _conv(xt, w_taps, bias, groups, H, W, act, BH=BH,
                halo=2)
    out = out.reshape(N, H, W, 2, 2, Cout).transpose(0, 5, 1, 3, 2, 4)
    return out.reshape(N, Cout, 2 * H, 2 * W)


_RQ_TILE = 1568


def _rq_body(z_ref, cb_ref, q_ref, i_ref, l_ref, *, Q, K, C):
    T = z_ref.shape[0]
    r = z_ref[...]
    qacc = jnp.zeros((T, C), jnp.float32)
    iota = lax.broadcasted_iota(jnp.int32, (T, K), 1)
    idx_cols = []
    loss_cols = []
    for q in range(Q):
        cb = cb_ref[q]
        cb2 = jnp.sum(cb ** 2, axis=-1)
        mm = lax.dot_general(
            r.astype(jnp.bfloat16), cb.astype(jnp.bfloat16),
            (((1,), (1,)), ((), ())),
            preferred_element_type=jnp.float32)
        d = jnp.sum(r ** 2, axis=-1, keepdims=True) - 2.0 * mm + cb2[None, :]
        dmin = jnp.min(d, axis=1, keepdims=True)
        masked = jnp.where(d <= dmin, iota, K)
        idx = jnp.min(masked, axis=1, keepdims=True)
        onehot = (iota == idx).astype(jnp.float32)
        qv = jnp.dot(onehot, cb, preferred_element_type=jnp.float32,
                     precision=lax.Precision.HIGHEST)
        diff = qv - r
        loss_cols.append(jnp.sum(diff * diff).reshape(1, 1))
        qv_st = r + diff
        r = r - qv_st
        qacc = qacc + qv_st
        idx_cols.append(idx)
    q_ref[...] = qacc
    i_ref[...] = jnp.concatenate(idx_cols, axis=1)
    lvec = jnp.concatenate(loss_cols, axis=1)

    @pl.when(pl.program_id(0) == 0)
    def _():
        l_ref[...] = jnp.zeros((1, Q), jnp.float32)

    l_ref[...] = l_ref[...] + lvec


def _rq(z_tokens, codebooks):
    M, C = z_tokens.shape
    Q, K, _ = codebooks.shape
    Tt = _RQ_TILE
    n_tiles = M // Tt
    body = functools.partial(_rq_body, Q=Q, K=K, C=C)
    qout, iout, lout = pl.pallas_call(
        body,
        grid=(n_tiles,),
        in_specs=[
            pl.BlockSpec((Tt, C), lambda n: (n, 0)),
            pl.BlockSpec((Q, K, C), lambda n: (0, 0, 0)),
        ],
        out_specs=[
            pl.BlockSpec((Tt, C), lambda n: (n, 0)),
            pl.BlockSpec((Tt, Q), lambda n: (n, 0)),
            pl.BlockSpec((1, Q), lambda n: (0, 0)),
        ],
        out_shape=[
            jax.ShapeDtypeStruct((M, C), jnp.float32),
            jax.ShapeDtypeStruct((M, Q), jnp.int32),
            jax.ShapeDtypeStruct((1, Q), jnp.float32),
        ],
    )(z_tokens, codebooks)
    return qout, iout, lout[0] / (M * C)


def kernel(x, enc_w0, enc_b0, enc_w1, enc_b1, enc_w2, enc_b2, enc_w3, enc_b3,
           codebooks, dec_w0, dec_b0, dec_w1, dec_b1, dec_w2, dec_b2, dec_w3,
           dec_b3):
    xn = x.transpose(0, 2, 3, 1)
    h = _conv_s2(xn, enc_w0, enc_b0, "relu")
    h = _conv_s2(h, enc_w1, enc_b1, "relu")
    h = _conv_s1(h, enc_w2, enc_b2, "relu")
    z = _conv_s1(h, enc_w3, enc_b3, "none")

    N, H, W, C = z.shape
    tokens = z.reshape(N * H * W, C)
    qtok, itok, commit_loss = _rq(tokens, codebooks)
    quantized_nhwc = qtok.reshape(N, H, W, C)
    indices = itok.reshape(N, H, W, codebooks.shape[0])
    quantized = quantized_nhwc.transpose(0, 3, 1, 2)

    d = _conv_s1(quantized_nhwc, dec_w0, dec_b0, "relu")
    d = _deconv_s2(d, dec_w1, dec_b1, "relu", BH=56)
    d = _deconv_s2(d, dec_w2, dec_b2, "relu", BH=28)
    recon = _deconv_s2_final(d, dec_w3, dec_b3, "tanh", BH=8)

    return (recon, indices, commit_loss, quantized)
